# baseline (speedup 1.0000x reference)
import functools

import jax
import jax.numpy as jnp
from jax import lax
from jax.experimental import pallas as pl
from jax.experimental.pallas import tpu as pltpu
from jax.experimental.pallas import tpu_sc as plsc

B = 4096
L = 200
D = 64
SCALE = float(D) ** 0.5
EPS = 1.1754943508222875e-38
N_ELEM = L * D

_NC = 2
_NS = 16
NW = _NC * _NS
BPW = B // NW
NBUF = 8

_PERM = [(m // 2) if m % 2 == 0 else 64 + (m // 2) for m in range(BPW)]
_INV_PERM = [0] * BPW
for _m, _k in enumerate(_PERM):
    _INV_PERM[_k] = _m

_mesh = plsc.VectorSubcoreMesh(core_axis_name="c", subcore_axis_name="s")


@functools.partial(
    pl.kernel,
    mesh=_mesh,
    compiler_params=pltpu.CompilerParams(use_tc_tiling_on_sc=False),
    out_type=[
        jax.ShapeDtypeStruct((L * B, D), jnp.float32),
        jax.ShapeDtypeStruct((B, D), jnp.float32),
        jax.ShapeDtypeStruct((B, D), jnp.float32),
    ],
    scratch_types=[
        pltpu.VMEM((L, BPW), jnp.int32),
        pltpu.VMEM((NBUF, BPW, D), jnp.float32),
        pltpu.VMEM((BPW, D), jnp.float32),
        pltpu.VMEM((BPW, D), jnp.float32),
        pltpu.SemaphoreType.DMA((NBUF,)),
        pltpu.SemaphoreType.DMA((NBUF,)),
    ],
)
def _sc_gather_stats(idx_hbm, emb_hbm, raw_hbm, s_hbm, q_hbm,
                     idx_v, rows_v, acc_s, acc_q, gsem, wsem):
    wid = lax.axis_index("s") * _NC + lax.axis_index("c")
    b0 = wid * BPW

    pltpu.sync_copy(idx_hbm.at[wid], idx_v)

    zeros = jnp.zeros((16,), jnp.float32)

    def zero_body(r, _):
        for c in range(D // 16):
            acc_s[r, pl.ds(c * 16, 16)] = zeros
            acc_q[r, pl.ds(c * 16, 16)] = zeros
        return 0
    lax.fori_loop(0, BPW, zero_body, 0)

    def fire_gather(l, j):
        pltpu.async_copy(emb_hbm.at[idx_v.at[l]], rows_v.at[j], gsem.at[j])

    def fire_write(l, j):
        pltpu.async_copy(rows_v.at[j], raw_hbm.at[pl.ds(l * B + b0, BPW)],
                         wsem.at[j])

    def wait_gather(j):
        pltpu.make_async_copy(emb_hbm.at[idx_v.at[0]], rows_v.at[j],
                              gsem.at[j]).wait()

    def wait_write(j):
        pltpu.make_async_copy(rows_v.at[j], raw_hbm.at[pl.ds(b0, BPW)],
                              wsem.at[j]).wait()

    def accumulate(j):
        def r_body(r, _):
            for c in range(D // 16):
                x = rows_v[j, r, pl.ds(c * 16, 16)]
                acc_s[r, pl.ds(c * 16, 16)] += x
                acc_q[r, pl.ds(c * 16, 16)] += x * x
            return 0
        lax.fori_loop(0, BPW, r_body, 0)

    def step(l, j, jprev, first):
        wait_gather(j)
        fire_write(l, j)
        accumulate(j)
        if not first:
            wait_write(jprev)
        fire_gather(l + NBUF - 1, jprev)

    for j in range(NBUF - 1):
        fire_gather(j, j)

    step(0, 0, NBUF - 1, True)
    for l in range(1, NBUF):
        step(l, l % NBUF, (l - 1) % NBUF, False)

    def chunk(c, _):
        base = c * NBUF
        for j in range(NBUF):
            step(base + j, j, (j - 1) % NBUF, False)
        return 0
    lax.fori_loop(1, (L // NBUF) - 1, chunk, 0)

    step(L - NBUF, (L - NBUF) % NBUF, (L - NBUF - 1) % NBUF, False)
    for l in range(L - NBUF + 1, L):
        wait_gather(l % NBUF)
        fire_write(l, l % NBUF)
        accumulate(l % NBUF)

    for j in range(NBUF):
        wait_write(j)

    pltpu.sync_copy(acc_s, s_hbm.at[pl.ds(b0, BPW)])
    pltpu.sync_copy(acc_q, q_hbm.at[pl.ds(b0, BPW)])


def _finalize_math(s, q):
    se = jnp.sum(s[:, :, :D], axis=2)
    so = jnp.sum(s[:, :, D:], axis=2)
    qe = jnp.sum(q[:, :, :D], axis=2)
    qo = jnp.sum(q[:, :, D:], axis=2)
    sum_b = jnp.concatenate([se, so], axis=-1)
    sumsq_b = jnp.concatenate([qe, qo], axis=-1)
    n = jnp.float32(N_ELEM)
    mean = sum_b / n
    var = (sumsq_b - sum_b * sum_b / n) / (n - 1.0)
    std = jnp.sqrt(var)
    inv = SCALE / (SCALE * std + EPS)
    return inv, -mean * inv

_LBLK = 4
_WGRP = 32


def _norm_body(x_ref, s_ref, q_ref, o_ref, a_scr, c_scr):
    @pl.when(pl.program_id(0) == 0)
    def _():
        a, c = _finalize_math(s_ref[...], q_ref[...])
        a_scr[...] = a
        c_scr[...] = c

    eye = jnp.eye(D, dtype=jnp.float32)
    for k in range(_WGRP):
        av = a_scr[k]
        cv = c_scr[k]
        for l in range(_LBLK):
            xl = x_ref[l, k]
            xh = xl.astype(jnp.bfloat16).astype(jnp.float32)
            xr = xl - xh
            dn = (((0,), (0,)), ((), ()))
            xt = lax.dot_general(
                xh, eye, dimension_numbers=dn,
                preferred_element_type=jnp.float32) + lax.dot_general(
                xr, eye, dimension_numbers=dn,
                preferred_element_type=jnp.float32)
            oc = jnp.concatenate([xt[:D, :], xt[D:, :]], axis=1)
            o_ref[l, :, pl.ds(k * 128, 128)] = oc * av + cv


_norm = pl.pallas_call(
    _norm_body,
    grid=(L // _LBLK,),
    in_specs=[
        pl.BlockSpec((_LBLK, _WGRP, D, BPW), lambda i: (i, 0, 0, 0)),
        pl.BlockSpec((NW, D, BPW), lambda i: (0, 0, 0)),
        pl.BlockSpec((NW, D, BPW), lambda i: (0, 0, 0)),
    ],
    out_specs=pl.BlockSpec((_LBLK, D, _WGRP * BPW), lambda i: (i, 0, 0)),
    out_shape=jax.ShapeDtypeStruct((L, D, B), jnp.float32),
    scratch_shapes=[
        pltpu.VMEM((NW, BPW), jnp.float32),
        pltpu.VMEM((NW, BPW), jnp.float32),
    ],
)


def kernel(inp, emb):
    perm = jnp.asarray(_PERM, dtype=jnp.int32)
    idx_w = inp.reshape(NW, BPW, L)[:, perm, :].transpose(0, 2, 1)
    raw, s, q = _sc_gather_stats(idx_w, emb)
    y = _norm(raw.reshape(L, NW, D, BPW),
              s.reshape(NW, D, BPW), q.reshape(NW, D, BPW))
    return jnp.transpose(y, (0, 2, 1))


# speedup vs baseline: 1.0023x; 1.0023x over previous
import functools

import jax
import jax.numpy as jnp
from jax import lax
from jax.experimental import pallas as pl
from jax.experimental.pallas import tpu as pltpu
from jax.experimental.pallas import tpu_sc as plsc

B = 4096
L = 200
D = 64
SCALE = float(D) ** 0.5
EPS = 1.1754943508222875e-38
N_ELEM = L * D

_NC = 2
_NS = 16
NW = _NC * _NS
BPW = B // NW
NBUF = 8

_PERM = [(m // 2) if m % 2 == 0 else 64 + (m // 2) for m in range(BPW)]
_INV_PERM = [0] * BPW
for _m, _k in enumerate(_PERM):
    _INV_PERM[_k] = _m

_mesh = plsc.VectorSubcoreMesh(core_axis_name="c", subcore_axis_name="s")


@functools.partial(
    pl.kernel,
    mesh=_mesh,
    compiler_params=pltpu.CompilerParams(use_tc_tiling_on_sc=False),
    out_type=[
        jax.ShapeDtypeStruct((L * B, D), jnp.float32),
        jax.ShapeDtypeStruct((B, D), jnp.float32),
        jax.ShapeDtypeStruct((B, D), jnp.float32),
    ],
    scratch_types=[
        pltpu.VMEM((L, BPW), jnp.int32),
        pltpu.VMEM((NBUF, BPW, D), jnp.float32),
        pltpu.VMEM((BPW, D), jnp.float32),
        pltpu.VMEM((BPW, D), jnp.float32),
        pltpu.SemaphoreType.DMA((NBUF,)),
        pltpu.SemaphoreType.DMA((NBUF,)),
    ],
)
def _sc_gather_stats(idx_hbm, emb_hbm, raw_hbm, s_hbm, q_hbm,
                     idx_v, rows_v, acc_s, acc_q, gsem, wsem):
    wid = lax.axis_index("s") * _NC + lax.axis_index("c")
    b0 = wid * BPW

    pltpu.sync_copy(idx_hbm.at[wid], idx_v)

    zeros = jnp.zeros((16,), jnp.float32)

    def zero_body(r, _):
        for c in range(D // 16):
            acc_s[r, pl.ds(c * 16, 16)] = zeros
            acc_q[r, pl.ds(c * 16, 16)] = zeros
        return 0
    lax.fori_loop(0, BPW, zero_body, 0)

    def fire_gather(l, j):
        pltpu.async_copy(emb_hbm.at[idx_v.at[l]], rows_v.at[j], gsem.at[j])

    def fire_write(l, j):
        pltpu.async_copy(rows_v.at[j], raw_hbm.at[pl.ds(l * B + b0, BPW)],
                         wsem.at[j])

    def wait_gather(j):
        pltpu.make_async_copy(emb_hbm.at[idx_v.at[0]], rows_v.at[j],
                              gsem.at[j]).wait()

    def wait_write(j):
        pltpu.make_async_copy(rows_v.at[j], raw_hbm.at[pl.ds(b0, BPW)],
                              wsem.at[j]).wait()

    def accumulate(j):
        def r_body(r, _):
            for c in range(D // 16):
                x = rows_v[j, r, pl.ds(c * 16, 16)]
                acc_s[r, pl.ds(c * 16, 16)] += x
                acc_q[r, pl.ds(c * 16, 16)] += x * x
            return 0
        lax.fori_loop(0, BPW, r_body, 0)

    def step(l, j, jprev, first):
        wait_gather(j)
        fire_write(l, j)
        accumulate(j)
        if not first:
            wait_write(jprev)
        fire_gather(l + NBUF - 1, jprev)

    for j in range(NBUF - 1):
        fire_gather(j, j)

    step(0, 0, NBUF - 1, True)
    for l in range(1, NBUF):
        step(l, l % NBUF, (l - 1) % NBUF, False)

    def chunk(c, _):
        base = c * NBUF
        for j in range(NBUF):
            step(base + j, j, (j - 1) % NBUF, False)
        return 0
    lax.fori_loop(1, (L // NBUF) - 1, chunk, 0)

    step(L - NBUF, (L - NBUF) % NBUF, (L - NBUF - 1) % NBUF, False)
    for l in range(L - NBUF + 1, L):
        wait_gather(l % NBUF)
        fire_write(l, l % NBUF)
        accumulate(l % NBUF)

    for j in range(NBUF):
        wait_write(j)

    pltpu.sync_copy(acc_s, s_hbm.at[pl.ds(b0, BPW)])
    pltpu.sync_copy(acc_q, q_hbm.at[pl.ds(b0, BPW)])


def _finalize_math(s, q):
    se = jnp.sum(s[:, :, :D], axis=2)
    so = jnp.sum(s[:, :, D:], axis=2)
    qe = jnp.sum(q[:, :, :D], axis=2)
    qo = jnp.sum(q[:, :, D:], axis=2)
    sum_b = jnp.concatenate([se, so], axis=-1)
    sumsq_b = jnp.concatenate([qe, qo], axis=-1)
    n = jnp.float32(N_ELEM)
    mean = sum_b / n
    var = (sumsq_b - sum_b * sum_b / n) / (n - 1.0)
    std = jnp.sqrt(var)
    inv = SCALE / (SCALE * std + EPS)
    return inv, -mean * inv

_LBLK = 8
_WGRP = 32


def _norm_body(x_ref, s_ref, q_ref, o_ref, a_scr, c_scr):
    @pl.when(pl.program_id(0) == 0)
    def _():
        a, c = _finalize_math(s_ref[...], q_ref[...])
        a_scr[...] = a
        c_scr[...] = c

    eye = jnp.eye(D, dtype=jnp.float32)
    for k in range(_WGRP):
        av = a_scr[k]
        cv = c_scr[k]
        for l in range(_LBLK):
            xl = x_ref[l, k]
            xh = xl.astype(jnp.bfloat16).astype(jnp.float32)
            xr = xl - xh
            dn = (((0,), (0,)), ((), ()))
            xt = lax.dot_general(
                xh, eye, dimension_numbers=dn,
                preferred_element_type=jnp.float32) + lax.dot_general(
                xr, eye, dimension_numbers=dn,
                preferred_element_type=jnp.float32)
            oc = jnp.concatenate([xt[:D, :], xt[D:, :]], axis=1)
            o_ref[l, :, pl.ds(k * 128, 128)] = oc * av + cv


_norm = pl.pallas_call(
    _norm_body,
    grid=(L // _LBLK,),
    in_specs=[
        pl.BlockSpec((_LBLK, _WGRP, D, BPW), lambda i: (i, 0, 0, 0)),
        pl.BlockSpec((NW, D, BPW), lambda i: (0, 0, 0)),
        pl.BlockSpec((NW, D, BPW), lambda i: (0, 0, 0)),
    ],
    out_specs=pl.BlockSpec((_LBLK, D, _WGRP * BPW), lambda i: (i, 0, 0)),
    out_shape=jax.ShapeDtypeStruct((L, D, B), jnp.float32),
    scratch_shapes=[
        pltpu.VMEM((NW, BPW), jnp.float32),
        pltpu.VMEM((NW, BPW), jnp.float32),
    ],
)


def kernel(inp, emb):
    perm = jnp.asarray(_PERM, dtype=jnp.int32)
    idx_w = inp.reshape(NW, BPW, L)[:, perm, :].transpose(0, 2, 1)
    raw, s, q = _sc_gather_stats(idx_w, emb)
    y = _norm(raw.reshape(L, NW, D, BPW),
              s.reshape(NW, D, BPW), q.reshape(NW, D, BPW))
    return jnp.transpose(y, (0, 2, 1))
